# baseline (speedup 1.0000x reference)
import functools

import jax
import jax.numpy as jnp
from jax.experimental import pallas as pl
from jax.experimental.pallas import tpu as pltpu

BLK = 128


def _body(rn_ref, x_ref, out_ref):
    i = pl.program_id(0)
    j = pl.program_id(1)
    nb = pl.num_programs(1)
    rn = rn_ref[i]
    mid = rn // 2
    first = mid // BLK
    last = jnp.maximum(first, (rn - 1) // BLK)

    @pl.when(j == 0)
    def _init():
        out_ref[...] = jnp.zeros_like(out_ref)

    @pl.when(j <= last - first)
    def _acc():
        jj = first + j
        row0 = jj * BLK
        rows = row0 + jax.lax.broadcasted_iota(jnp.int32, (1, BLK), 1)
        maskf = ((rows >= mid) & (rows < rn)).astype(jnp.float32)
        blk = x_ref[0]
        partial = jax.lax.dot(maskf, blk,
                              preferred_element_type=jnp.float32)
        out_ref[0] += partial

    @pl.when(j == nb - 1)
    def _fin():
        count = (rn - mid).astype(jnp.float32)
        out_ref[...] = out_ref[...] / count


def _x_index(i, j, rn_ref):
    rn = rn_ref[i]
    mid = rn // 2
    first = mid // BLK
    last = jnp.maximum(first, (rn - 1) // BLK)
    jj = first + jnp.minimum(j, last - first)
    return (i, jj, 0)


def kernel(x, record_num):
    B, L, D = x.shape
    nb = L // BLK
    grid_spec = pltpu.PrefetchScalarGridSpec(
        num_scalar_prefetch=1,
        grid=(B, nb),
        in_specs=[pl.BlockSpec((1, BLK, D), _x_index)],
        out_specs=pl.BlockSpec((1, 1, D), lambda i, j, rn_ref: (i, 0, 0)),
    )
    out = pl.pallas_call(
        _body,
        grid_spec=grid_spec,
        out_shape=jax.ShapeDtypeStruct((B, 1, D), jnp.float32),
    )(record_num.astype(jnp.int32), x)
    return out.reshape(B, D)
